# baseline (speedup 1.0000x reference)
import jax
import jax.numpy as jnp
from jax import lax
from jax.experimental import pallas as pl
from jax.experimental.pallas import tpu as pltpu

_NB = 1664
_TOPK = 1000
_CONF = 0.05
_NMS_T = 0.6
_NEG = -3.0e38
_BIGI = 1 << 30
_INTERPRET = False


def _yolof_body(s_in, box_in, x1o, y1o, x2o, y2o, sco, lbo, kpo, s_scr,
                x1t, y1t, x2t, y2t, iou_scr, fl_smem):
    i32 = jnp.int32
    fio = (lax.broadcasted_iota(i32, (8, 128), 0) * 128
           + lax.broadcasted_iota(i32, (8, 128), 1))
    bio = (lax.broadcasted_iota(i32, (16, 128), 0) * 128
           + lax.broadcasted_iota(i32, (16, 128), 1))

    def _cb(k, bm):
        v = s_in[pl.ds(k * 8, 8)]
        s_scr[pl.ds(k * 8, 8)] = v
        for j in range(8):
            bm = jnp.where(bio == k * 8 + j, jnp.max(v[j]), bm)
        return bm

    bm0 = lax.fori_loop(0, _NB // 8, _cb,
                        jnp.full((16, 128), -1.0, jnp.float32))

    zf = jnp.zeros((8, 128), jnp.float32)
    zi = jnp.zeros((8, 128), i32)

    def _ext(i, carry):
        bm, sca = carry
        m = jnp.max(bm)
        b = jnp.min(jnp.where(bm == m, bio, _BIGI))
        blk = s_scr[b]
        j = jnp.min(jnp.where(blk == m, fio, _BIGI))
        fl_smem[i] = b * 1024 + j
        sca = jnp.where(fio == i, m, sca)
        nblk = jnp.where(fio == j, -1.0, blk)
        s_scr[b] = nblk
        bm = jnp.where(bio == b, jnp.max(nblk), bm)
        return bm, sca

    _, sc = lax.fori_loop(0, _TOPK, _ext, (bm0, zf))

    def _gat(i, carry):
        x1a, y1a, x2a, y2a, lba = carry
        flat = fl_smem[i]
        anchor = flat // 80
        label = flat % 80
        row = box_in[pl.ds(anchor, 1), :]
        sel = fio == i
        x1a = jnp.where(sel, row[:, 0:1], x1a)
        y1a = jnp.where(sel, row[:, 1:2], y1a)
        x2a = jnp.where(sel, row[:, 2:3], x2a)
        y2a = jnp.where(sel, row[:, 3:4], y2a)
        lba = jnp.where(sel, label, lba)
        return x1a, y1a, x2a, y2a, lba

    x1, y1, x2, y2, lb = lax.fori_loop(
        0, _TOPK, _gat, (zf, zf, zf, zf, zi))

    lbf = lb.astype(jnp.float32)
    maxc = jnp.maximum(jnp.maximum(jnp.max(x1), jnp.max(y1)),
                       jnp.maximum(jnp.max(x2), jnp.max(y2)))
    off = lbf * (maxc + 1.0)
    ox1 = x1 + off
    oy1 = y1 + off
    ox2 = x2 + off
    oy2 = y2 + off
    area = jnp.maximum(ox2 - ox1, 0.0) * jnp.maximum(oy2 - oy1, 0.0)
    keep0 = (sc > _CONF).astype(jnp.float32)

    x1t[...] = jnp.transpose(ox1)
    y1t[...] = jnp.transpose(oy1)
    x2t[...] = jnp.transpose(ox2)
    y2t[...] = jnp.transpose(oy2)

    for ri in range(8):
        def _iou_blk(cb, _, ri=ri):
            x1b = x1t[pl.ds(cb * 8, 8), ri:ri + 1].reshape(8, 1, 1)
            y1b = y1t[pl.ds(cb * 8, 8), ri:ri + 1].reshape(8, 1, 1)
            x2b = x2t[pl.ds(cb * 8, 8), ri:ri + 1].reshape(8, 1, 1)
            y2b = y2t[pl.ds(cb * 8, 8), ri:ri + 1].reshape(8, 1, 1)
            ab = jnp.maximum(x2b - x1b, 0.0) * jnp.maximum(y2b - y1b, 0.0)
            ix1 = jnp.maximum(ox1[None], x1b)
            iy1 = jnp.maximum(oy1[None], y1b)
            ix2 = jnp.minimum(ox2[None], x2b)
            iy2 = jnp.minimum(oy2[None], y2b)
            inter = (jnp.maximum(ix2 - ix1, 0.0)
                     * jnp.maximum(iy2 - iy1, 0.0))
            union = ab + area[None] - inter
            iou_scr[pl.ds(ri * 128 + cb * 8, 8)] = (
                inter / jnp.maximum(union, 1e-10))
            return _

        lax.fori_loop(0, 16, _iou_blk, 0)

    def _nms8(g, kv):
        blk = iou_scr[pl.ds(g * 8, 8)]
        for r in range(8):
            i = g * 8 + r
            row = blk[r]
            ki = jnp.max(jnp.where(fio == i, kv, 0.0))
            sup = (row > _NMS_T) & (fio > i) & (ki > 0.5)
            kv = jnp.where(sup, 0.0, kv)
        return kv

    kf = lax.fori_loop(0, _TOPK // 8, _nms8, keep0)

    lbo[...] = lb
    kpo[...] = kf
    sco[...] = sc * kf
    x1o[...] = x1 * kf
    y1o[...] = y1 * kf
    x2o[...] = x2 * kf
    y2o[...] = y2 * kf


def kernel(cls_pred, box_pred):
    cls = cls_pred[0]
    box = box_pred[0]
    scores = jax.nn.sigmoid(cls).reshape(-1)
    pad = _NB * 1024 - scores.shape[0]
    s3 = jnp.concatenate(
        [scores, jnp.full((pad,), -1.0, jnp.float32)]).reshape(_NB, 8, 128)
    o = jax.ShapeDtypeStruct((8, 128), jnp.float32)
    oi = jax.ShapeDtypeStruct((8, 128), jnp.int32)
    x1, y1, x2, y2, sc, lb, kp = pl.pallas_call(
        _yolof_body,
        out_shape=[o, o, o, o, o, oi, o],
        scratch_shapes=[
            pltpu.VMEM((_NB, 8, 128), jnp.float32),
            pltpu.VMEM((128, 8), jnp.float32),
            pltpu.VMEM((128, 8), jnp.float32),
            pltpu.VMEM((128, 8), jnp.float32),
            pltpu.VMEM((128, 8), jnp.float32),
            pltpu.VMEM((1024, 8, 128), jnp.float32),
            pltpu.SMEM((1024,), jnp.int32),
        ],
        interpret=_INTERPRET,
    )(s3, box)
    bboxes = jnp.stack([x1.reshape(-1), y1.reshape(-1),
                        x2.reshape(-1), y2.reshape(-1)], axis=-1)[:_TOPK]
    scores_out = sc.reshape(-1)[:_TOPK]
    labels = lb.reshape(-1)[:_TOPK]
    keep = kp.reshape(-1)[:_TOPK] > 0.5
    return bboxes, scores_out, labels, keep


# speedup vs baseline: 1.1315x; 1.1315x over previous
import jax
import jax.numpy as jnp
from jax import lax
from jax.experimental import pallas as pl
from jax.experimental.pallas import tpu as pltpu

_NB = 1664
_TOPK = 1000
_CONF = 0.05
_NMS_T = 0.6
_BIGI = 1 << 30


def _yolof_body(s_in, box_in, x1o, y1o, x2o, y2o, sco, lbo, kpo, s_scr,
                x1t, y1t, x2t, y2t, iou_scr):
    i32 = jnp.int32
    fio = (lax.broadcasted_iota(i32, (8, 128), 0) * 128
           + lax.broadcasted_iota(i32, (8, 128), 1))
    bio = (lax.broadcasted_iota(i32, (16, 128), 0) * 128
           + lax.broadcasted_iota(i32, (16, 128), 1))

    def _cb(k, bm):
        v = s_in[pl.ds(k * 8, 8)]
        s_scr[pl.ds(k * 8, 8)] = v
        for j in range(8):
            bm = jnp.where(bio == k * 8 + j, jnp.max(v[j]), bm)
        return bm

    bm0 = lax.fori_loop(0, _NB // 8, _cb,
                        jnp.full((16, 128), -1.0, jnp.float32))

    zf = jnp.zeros((8, 128), jnp.float32)
    zi = jnp.zeros((8, 128), i32)

    def _ext(i, carry):
        bm, x1a, y1a, x2a, y2a, sca, lba = carry
        m = jnp.max(bm)
        b = jnp.min(jnp.where(bm == m, bio, _BIGI))
        blk = s_scr[b]
        j = jnp.min(jnp.where(blk == m, fio, _BIGI))
        flat = b * 1024 + j
        anchor = flat // 80
        label = flat % 80
        row = box_in[pl.ds(anchor, 1), :]
        sel = fio == i
        x1a = jnp.where(sel, row[:, 0:1], x1a)
        y1a = jnp.where(sel, row[:, 1:2], y1a)
        x2a = jnp.where(sel, row[:, 2:3], x2a)
        y2a = jnp.where(sel, row[:, 3:4], y2a)
        sca = jnp.where(sel, m, sca)
        lba = jnp.where(sel, label, lba)
        nblk = jnp.where(fio == j, -1.0, blk)
        s_scr[b] = nblk
        bm = jnp.where(bio == b, jnp.max(nblk), bm)
        return bm, x1a, y1a, x2a, y2a, sca, lba

    _, x1, y1, x2, y2, sc, lb = lax.fori_loop(
        0, _TOPK, _ext, (bm0, zf, zf, zf, zf, zf, zi))

    lbf = lb.astype(jnp.float32)
    maxc = jnp.maximum(jnp.maximum(jnp.max(x1), jnp.max(y1)),
                       jnp.maximum(jnp.max(x2), jnp.max(y2)))
    off = lbf * (maxc + 1.0)
    ox1 = x1 + off
    oy1 = y1 + off
    ox2 = x2 + off
    oy2 = y2 + off
    area = jnp.maximum(ox2 - ox1, 0.0) * jnp.maximum(oy2 - oy1, 0.0)
    keep0 = (sc > _CONF).astype(jnp.float32)

    x1t[...] = jnp.transpose(ox1)
    y1t[...] = jnp.transpose(oy1)
    x2t[...] = jnp.transpose(ox2)
    y2t[...] = jnp.transpose(oy2)

    for ri in range(8):
        def _iou_blk(cb, _, ri=ri):
            x1b = x1t[pl.ds(cb * 8, 8), ri:ri + 1].reshape(8, 1, 1)
            y1b = y1t[pl.ds(cb * 8, 8), ri:ri + 1].reshape(8, 1, 1)
            x2b = x2t[pl.ds(cb * 8, 8), ri:ri + 1].reshape(8, 1, 1)
            y2b = y2t[pl.ds(cb * 8, 8), ri:ri + 1].reshape(8, 1, 1)
            ab = jnp.maximum(x2b - x1b, 0.0) * jnp.maximum(y2b - y1b, 0.0)
            ix1 = jnp.maximum(ox1[None], x1b)
            iy1 = jnp.maximum(oy1[None], y1b)
            ix2 = jnp.minimum(ox2[None], x2b)
            iy2 = jnp.minimum(oy2[None], y2b)
            inter = (jnp.maximum(ix2 - ix1, 0.0)
                     * jnp.maximum(iy2 - iy1, 0.0))
            union = ab + area[None] - inter
            iou_scr[pl.ds(ri * 128 + cb * 8, 8)] = (
                inter / jnp.maximum(union, 1e-10))
            return _

        lax.fori_loop(0, 16, _iou_blk, 0)

    def _nms8(g, kv):
        blk = iou_scr[pl.ds(g * 8, 8)]
        for r in range(8):
            i = g * 8 + r
            row = blk[r]
            ki = jnp.max(jnp.where(fio == i, kv, 0.0))
            sup = (row > _NMS_T) & (fio > i) & (ki > 0.5)
            kv = jnp.where(sup, 0.0, kv)
        return kv

    kf = lax.fori_loop(0, _TOPK // 8, _nms8, keep0)

    lbo[...] = lb
    kpo[...] = kf
    sco[...] = sc * kf
    x1o[...] = x1 * kf
    y1o[...] = y1 * kf
    x2o[...] = x2 * kf
    y2o[...] = y2 * kf


def kernel(cls_pred, box_pred):
    cls = cls_pred[0]
    box = box_pred[0]
    scores = jax.nn.sigmoid(cls).reshape(-1)
    pad = _NB * 1024 - scores.shape[0]
    s3 = jnp.concatenate(
        [scores, jnp.full((pad,), -1.0, jnp.float32)]).reshape(_NB, 8, 128)
    o = jax.ShapeDtypeStruct((8, 128), jnp.float32)
    oi = jax.ShapeDtypeStruct((8, 128), jnp.int32)
    x1, y1, x2, y2, sc, lb, kp = pl.pallas_call(
        _yolof_body,
        out_shape=[o, o, o, o, o, oi, o],
        scratch_shapes=[
            pltpu.VMEM((_NB, 8, 128), jnp.float32),
            pltpu.VMEM((128, 8), jnp.float32),
            pltpu.VMEM((128, 8), jnp.float32),
            pltpu.VMEM((128, 8), jnp.float32),
            pltpu.VMEM((128, 8), jnp.float32),
            pltpu.VMEM((1024, 8, 128), jnp.float32),
        ],
    )(s3, box)
    bboxes = jnp.stack([x1.reshape(-1), y1.reshape(-1),
                        x2.reshape(-1), y2.reshape(-1)], axis=-1)[:_TOPK]
    scores_out = sc.reshape(-1)[:_TOPK]
    labels = lb.reshape(-1)[:_TOPK]
    keep = kp.reshape(-1)[:_TOPK] > 0.5
    return bboxes, scores_out, labels, keep


# speedup vs baseline: 1.1418x; 1.0091x over previous
import jax
import jax.numpy as jnp
from jax import lax
from jax.experimental import pallas as pl
from jax.experimental.pallas import tpu as pltpu

_NB = 1664
_TOPK = 1000
_CONF = 0.05
_NMS_T = 0.6
_BIGI = 1 << 30


def _yolof_body(s_in, box_in, x1o, y1o, x2o, y2o, sco, lbo, kpo,
                x1t, y1t, x2t, y2t, iou_scr):
    i32 = jnp.int32
    fio = (lax.broadcasted_iota(i32, (8, 128), 0) * 128
           + lax.broadcasted_iota(i32, (8, 128), 1))
    bio = (lax.broadcasted_iota(i32, (16, 128), 0) * 128
           + lax.broadcasted_iota(i32, (16, 128), 1))

    def _cb(k, bm):
        v = s_in[pl.ds(k * 8, 8)]
        for j in range(8):
            bm = jnp.where(bio == k * 8 + j, jnp.max(v[j]), bm)
        return bm

    bm0 = lax.fori_loop(0, _NB // 8, _cb,
                        jnp.full((16, 128), -1.0, jnp.float32))

    zf = jnp.zeros((8, 128), jnp.float32)
    zi = jnp.zeros((8, 128), i32)

    def _ext(i, carry):
        bm, x1a, y1a, x2a, y2a, sca, lba = carry
        m = jnp.max(bm)
        b = jnp.min(jnp.where(bm == m, bio, _BIGI))
        blk = s_in[b]
        j = jnp.min(jnp.where(blk == m, fio, _BIGI))
        flat = b * 1024 + j
        anchor = flat // 80
        label = flat % 80
        row = box_in[pl.ds(anchor, 1), :]
        sel = fio == i
        x1a = jnp.where(sel, row[:, 0:1], x1a)
        y1a = jnp.where(sel, row[:, 1:2], y1a)
        x2a = jnp.where(sel, row[:, 2:3], x2a)
        y2a = jnp.where(sel, row[:, 3:4], y2a)
        sca = jnp.where(sel, m, sca)
        lba = jnp.where(sel, label, lba)
        nblk = jnp.where(fio == j, -1.0, blk)
        s_in[b] = nblk
        bm = jnp.where(bio == b, jnp.max(nblk), bm)
        return bm, x1a, y1a, x2a, y2a, sca, lba

    _, x1, y1, x2, y2, sc, lb = lax.fori_loop(
        0, _TOPK, _ext, (bm0, zf, zf, zf, zf, zf, zi))

    lbf = lb.astype(jnp.float32)
    maxc = jnp.maximum(jnp.maximum(jnp.max(x1), jnp.max(y1)),
                       jnp.maximum(jnp.max(x2), jnp.max(y2)))
    off = lbf * (maxc + 1.0)
    ox1 = x1 + off
    oy1 = y1 + off
    ox2 = x2 + off
    oy2 = y2 + off
    area = jnp.maximum(ox2 - ox1, 0.0) * jnp.maximum(oy2 - oy1, 0.0)
    keep0 = (sc > _CONF).astype(jnp.float32)

    x1t[...] = jnp.transpose(ox1)
    y1t[...] = jnp.transpose(oy1)
    x2t[...] = jnp.transpose(ox2)
    y2t[...] = jnp.transpose(oy2)

    for ri in range(8):
        def _iou_blk(cb, _, ri=ri):
            x1b = x1t[pl.ds(cb * 8, 8), ri:ri + 1].reshape(8, 1, 1)
            y1b = y1t[pl.ds(cb * 8, 8), ri:ri + 1].reshape(8, 1, 1)
            x2b = x2t[pl.ds(cb * 8, 8), ri:ri + 1].reshape(8, 1, 1)
            y2b = y2t[pl.ds(cb * 8, 8), ri:ri + 1].reshape(8, 1, 1)
            ab = jnp.maximum(x2b - x1b, 0.0) * jnp.maximum(y2b - y1b, 0.0)
            ix1 = jnp.maximum(ox1[None], x1b)
            iy1 = jnp.maximum(oy1[None], y1b)
            ix2 = jnp.minimum(ox2[None], x2b)
            iy2 = jnp.minimum(oy2[None], y2b)
            inter = (jnp.maximum(ix2 - ix1, 0.0)
                     * jnp.maximum(iy2 - iy1, 0.0))
            union = ab + area[None] - inter
            iou_scr[pl.ds(ri * 128 + cb * 8, 8)] = (
                inter / jnp.maximum(union, 1e-10))
            return _

        lax.fori_loop(0, 16, _iou_blk, 0)

    def _nms8(g, kv):
        blk = iou_scr[pl.ds(g * 8, 8)]
        for r in range(8):
            i = g * 8 + r
            row = blk[r]
            ki = jnp.max(jnp.where(fio == i, kv, 0.0))
            sup = (row > _NMS_T) & (fio > i) & (ki > 0.5)
            kv = jnp.where(sup, 0.0, kv)
        return kv

    kf = lax.fori_loop(0, _TOPK // 8, _nms8, keep0)

    lbo[...] = lb
    kpo[...] = kf
    sco[...] = sc * kf
    x1o[...] = x1 * kf
    y1o[...] = y1 * kf
    x2o[...] = x2 * kf
    y2o[...] = y2 * kf


def kernel(cls_pred, box_pred):
    cls = cls_pred[0]
    box = box_pred[0]
    scores = jax.nn.sigmoid(cls).reshape(-1)
    pad = _NB * 1024 - scores.shape[0]
    s3 = jnp.concatenate(
        [scores, jnp.full((pad,), -1.0, jnp.float32)]).reshape(_NB, 8, 128)
    o = jax.ShapeDtypeStruct((8, 128), jnp.float32)
    oi = jax.ShapeDtypeStruct((8, 128), jnp.int32)
    x1, y1, x2, y2, sc, lb, kp = pl.pallas_call(
        _yolof_body,
        out_shape=[o, o, o, o, o, oi, o],
        scratch_shapes=[
            pltpu.VMEM((128, 8), jnp.float32),
            pltpu.VMEM((128, 8), jnp.float32),
            pltpu.VMEM((128, 8), jnp.float32),
            pltpu.VMEM((128, 8), jnp.float32),
            pltpu.VMEM((1024, 8, 128), jnp.float32),
        ],
    )(s3, box)
    bboxes = jnp.stack([x1.reshape(-1), y1.reshape(-1),
                        x2.reshape(-1), y2.reshape(-1)], axis=-1)[:_TOPK]
    scores_out = sc.reshape(-1)[:_TOPK]
    labels = lb.reshape(-1)[:_TOPK]
    keep = kp.reshape(-1)[:_TOPK] > 0.5
    return bboxes, scores_out, labels, keep
